# baseline (speedup 1.0000x reference)
import functools
import jax
import jax.numpy as jnp
from jax import lax
from jax.experimental import pallas as pl
from jax.experimental.pallas import tpu as pltpu
from jax.experimental.pallas import tpu_sc as plsc

NC = 2
NS = 16
LANES = 16
NW = NC * NS
NSET = 2
JUN = 8


def _emb_call(ids_flat, token_table, pos_table, *, batch, seq, chunk):
    d = token_table.shape[1]
    d_vecs = d // LANES
    ppw = seq // NW
    n_groups = ppw // chunk

    mesh = plsc.VectorSubcoreMesh(core_axis_name="c", subcore_axis_name="s")

    @functools.partial(
        pl.kernel,
        out_type=jax.ShapeDtypeStruct((batch * seq, d), jnp.float32),
        mesh=mesh,
        scratch_types=[
            pltpu.VMEM((batch, ppw), jnp.int32),
            [[pltpu.VMEM((chunk, d), jnp.float32) for _ in range(batch)]
             for _ in range(NSET)],
            [pltpu.VMEM((chunk, d), jnp.float32) for _ in range(NSET)],
            [[pltpu.SemaphoreType.DMA for _ in range(batch)]
             for _ in range(NSET)],
            [pltpu.SemaphoreType.DMA for _ in range(NSET)],
            [[pltpu.SemaphoreType.DMA for _ in range(batch)]
             for _ in range(NSET)],
        ],
    )
    def k(ids_hbm, tok_hbm, pos_hbm, out_hbm, idx_v, rows, pos, gsem, psem, osem):
        wid = lax.axis_index("s") * NC + lax.axis_index("c")
        wpos = wid * ppw
        gather_h = {}
        store_h = {}
        pos_h = [None, None]

        def start_pos(g):
            s = g % NSET
            pos_h[s] = pltpu.async_copy(
                pos_hbm.at[pl.ds(wpos + g * chunk, chunk), :], pos[s], psem[s])

        def start_gathers(g):
            s = g % NSET
            for b in range(batch):
                gather_h[(g, b)] = pltpu.async_copy(
                    tok_hbm.at[idx_v.at[b, pl.ds(g * chunk, chunk)]],
                    rows[s][b], gsem[s][b])

        def start_stores(g):
            s = g % NSET
            store_h[g] = [
                pltpu.async_copy(
                    rows[s][b],
                    out_hbm.at[pl.ds(b * seq + wpos + g * chunk, chunk), :],
                    osem[s][b])
                for b in range(batch)]

        start_pos(0)
        start_pos(1)
        idx_h = [pltpu.async_copy(ids_hbm.at[pl.ds(b * seq + wpos, ppw)],
                                  idx_v.at[b], osem[1][0])
                 for b in range(batch)]
        for h in idx_h:
            h.wait()
        start_gathers(0)

        for g in range(n_groups):
            s = g % NSET
            if g >= 1:
                for h in store_h.pop(g - 1):
                    h.wait()
            if g + 1 < n_groups:
                start_gathers(g + 1)
            for b in range(batch):
                gather_h.pop((g, b)).wait()
            pos_h[s].wait()
            pbuf = pos[s]
            rbufs = rows[s]

            @plsc.parallel_loop(0, chunk, 1)
            def _(i):
                def j_body(j8, _):
                    for jj in range(JUN):
                        sl = pl.ds((j8 * JUN + jj) * LANES, LANES)
                        x = pbuf[i, sl]
                        for b in range(batch):
                            plsc.addupdate(rbufs[b].at[i, sl], x)
                    return 0
                lax.fori_loop(0, d_vecs // JUN, j_body, 0)

            start_stores(g)
            if g + NSET < n_groups:
                start_pos(g + NSET)
        for g in sorted(store_h):
            for h in store_h[g]:
                h.wait()

    return k(ids_flat, token_table, pos_table)


def kernel(input_ids, token_table, pos_table):
    b, s = input_ids.shape
    d = token_table.shape[1]
    ids_flat = input_ids.reshape(-1).astype(jnp.int32)
    out = _emb_call(ids_flat, token_table, pos_table, batch=b, seq=s, chunk=16)
    return out.reshape(b, s, d)
